# baseline (speedup 1.0000x reference)
import functools

import jax
import jax.numpy as jnp
from jax import lax
from jax.experimental import pallas as pl
from jax.experimental.pallas import tpu as pltpu
from jax.experimental.pallas import tpu_sc as plsc

_ND = 32
_NC = 2
_NS = 16
_NW = _NC * _NS
_L = 16


def _encode(idx_flat, table2, batch, k):
    bpw = batch // _NW
    ipw = bpw * k

    mesh = plsc.VectorSubcoreMesh(core_axis_name="c", subcore_axis_name="s")

    @functools.partial(
        pl.kernel,
        out_type=jax.ShapeDtypeStruct((batch, k * _ND), jnp.float32),
        mesh=mesh,
        scratch_types=[
            pltpu.VMEM((ipw,), jnp.int32),
            pltpu.VMEM((bpw,), jnp.int32),
            pltpu.VMEM((bpw,), jnp.int32),
            pltpu.VMEM((bpw, k * _ND), jnp.float32),
            pltpu.VMEM_SHARED((32 * _L, k * _ND), jnp.float32),
            pltpu.SemaphoreType.DMA,
            pltpu.SemaphoreType.DMA,
        ],
        compiler_params=pltpu.CompilerParams(
            use_tc_tiling_on_sc=True, needs_layout_passes=False
        ),
    )
    def run(idx_hbm, t2_hbm, out_hbm, raw_v, i01_v, i23_v, rows_v, t2_s,
            gsem, wsem):
        wid = lax.axis_index("s") * _NC + lax.axis_index("c")

        @pl.when(lax.axis_index("s") == 0)
        def _stage_table():
            pltpu.sync_copy(t2_hbm, t2_s)

        pltpu.sync_copy(idx_hbm.at[pl.ds(wid * ipw, ipw)], raw_v)

        lanes = lax.iota(jnp.int32, _L)
        four = lanes * 4
        for g in range(bpw // _L):
            base = g * _L * 4
            a0 = plsc.load_gather(raw_v, [four + base])
            a1 = plsc.load_gather(raw_v, [four + (base + 1)])
            a2 = plsc.load_gather(raw_v, [four + (base + 2)])
            a3 = plsc.load_gather(raw_v, [four + (base + 3)])
            i01_v[pl.ds(g * _L, _L)] = a0 + a1 * 16
            i23_v[pl.ds(g * _L, _L)] = a2 + a3 * 16 + 256

        plsc.subcore_barrier()
        pltpu.async_copy(t2_s.at[i01_v], rows_v, gsem).wait()
        pltpu.async_copy(t2_s.at[i23_v], rows_v, gsem, add=True).wait()
        pltpu.sync_copy(rows_v, out_hbm.at[pl.ds(wid * bpw, bpw)])

    return run(idx_flat, table2)


def kernel(addr_nibbles, nibble_basis):
    b, k = addr_nibbles.shape
    nd = nibble_basis.shape[1]
    ii = jnp.arange(256)
    lo = nibble_basis[ii % 16]
    hi = nibble_basis[ii // 16]
    z = jnp.zeros((256, 2 * nd), jnp.float32)
    t2 = jnp.concatenate(
        [
            jnp.concatenate([lo, hi, z], axis=1),
            jnp.concatenate([z, lo, hi], axis=1),
        ],
        axis=0,
    )
    idx_flat = addr_nibbles.astype(jnp.int32).reshape(-1)
    return _encode(idx_flat, t2, b, k)
